# Initial kernel scaffold; baseline (speedup 1.0000x reference)
import jax
import jax.numpy as jnp
from jax.experimental import pallas as pl


def kernel(constraint_features, edge_indices, edge_features, variable_features, variable_features_batch, cons_shift, cons_scale, W_c1, b_c1, W_c2, b_c2, edge_shift, edge_scale, W_e0, b_e0, var_shift, var_scale, W_v1, b_v1, W_v2, b_v2, g1_Wl, g1_bl, g1_Wr, g1_br, g1_We, g1_be, g1_att, g1_Wout, g1_bout, g2_Wl, g2_bl, g2_Wr, g2_br, g2_We, g2_be, g2_att, g2_Wout, g2_bout, W_o1, b_o1, W_o2):
    raise NotImplementedError("write your pallas kernel here")


# speedup vs baseline: 3.6367x; 3.6367x over previous
import functools

import jax
import jax.numpy as jnp
from jax import lax
from jax.experimental import pallas as pl
from jax.experimental.pallas import tpu as pltpu
from jax.experimental.pallas import tpu_sc as plsc

N = 25000
D = 64
E = 800000
LN = 16
NC_SC = 2
NS_SC = 16
NW = NC_SC * NS_SC
RPT = 196
EPT = RPT * 128
E_PAD = EPT * NW
EROWS = E_PAD // 128
CHUNK = 4
NCHUNKS = RPT // CHUNK
CE = CHUNK * 128
SP_ROWS = 25024
STRIPE = SP_ROWS // NS_SC

_mesh = plsc.VectorSubcoreMesh(
    core_axis_name="c", subcore_axis_name="s",
    num_cores=NC_SC, num_subcores=NS_SC)

_sc_params = pltpu.CompilerParams(
    needs_layout_passes=False, use_tc_tiling_on_sc=False)


def _wid():
    return lax.axis_index("s") * NC_SC + lax.axis_index("c")


def _logits_body(Lr, Rr, dst2, src2, ef1, consts, logits_o, max_o,
                 idxd, idxs, efv, lgv, ld, rd, cv, mxv, sem):
    wid = _wid()
    pltpu.sync_copy(consts, cv)
    li = lax.iota(jnp.int32, LN)
    u = [cv[0, pl.ds(q * LN, LN)] for q in range(4)]
    cc = [cv[1, pl.ds(q * LN, LN)] for q in range(4)]
    at = [cv[2, pl.ds(q * LN, LN)] for q in range(4)]

    def chunk_body(ch, gmx):
        row0 = wid * RPT + ch * CHUNK
        base = row0 * 128
        pltpu.sync_copy(dst2.at[pl.ds(row0, CHUNK)], idxd)
        pltpu.sync_copy(src2.at[pl.ds(row0, CHUNK)], idxs)
        pltpu.sync_copy(ef1.at[pl.ds(base, CE)], efv)
        cps = []
        for j in range(CHUNK):
            cps.append(pltpu.async_copy(
                Lr.at[idxd.at[j]], ld.at[pl.ds(j * 128, 128)], sem))
            cps.append(pltpu.async_copy(
                Rr.at[idxs.at[j]], rd.at[pl.ds(j * 128, 128)], sem))
        for cp in cps:
            cp.wait()

        def grp_body(g, gmx2):
            e0 = g * LN
            eiv = e0 + li
            fv = efv[pl.ds(e0, LN)]
            acc = jnp.zeros((LN,), jnp.float32)
            for f in range(D):
                colf = jnp.full((LN,), f, jnp.int32)
                lg = plsc.load_gather(ld, [eiv, colf])
                rg = plsc.load_gather(rd, [eiv, colf])
                q, m = divmod(f, LN)
                v = lg + rg + fv * u[q][m] + cc[q][m]
                v = jnp.maximum(v, 0.2 * v)
                acc = acc + v * at[q][m]
            eidv = base + e0 + li
            lvec = jnp.where(eidv < E, acc, -1e30)
            lgv[pl.ds(e0, LN)] = lvec
            return jnp.maximum(gmx2, lvec)

        gmx = lax.fori_loop(0, CE // LN, grp_body, gmx)
        pltpu.sync_copy(lgv, logits_o.at[pl.ds(base, CE)])
        return gmx

    gmx = lax.fori_loop(0, NCHUNKS, chunk_body,
                        jnp.full((LN,), -3e38, jnp.float32))
    mxv[pl.ds(0, LN)] = gmx
    pltpu.sync_copy(mxv, max_o.at[wid])


_logits_call = pl.kernel(
    _logits_body,
    out_type=(jax.ShapeDtypeStruct((E_PAD,), jnp.float32),
              jax.ShapeDtypeStruct((NW, LN), jnp.float32)),
    mesh=_mesh,
    compiler_params=_sc_params,
    scratch_types=(
        pltpu.VMEM((CHUNK, 128), jnp.int32),
        pltpu.VMEM((CHUNK, 128), jnp.int32),
        pltpu.VMEM((CE,), jnp.float32),
        pltpu.VMEM((CE,), jnp.float32),
        pltpu.VMEM((CE, D), jnp.float32),
        pltpu.VMEM((CE, D), jnp.float32),
        pltpu.VMEM((3, D), jnp.float32),
        pltpu.VMEM((LN,), jnp.float32),
        pltpu.SemaphoreType.DMA,
    ),
)


HD = D // 2
RPT2 = RPT * 2
NCHUNKS2 = RPT2 // CHUNK


def _agg_body(Rh, dst2, src2, ef1, logits_i, gmax_i, consts,
              num_o, den_o,
              idxd, idxs, efv, lgv, rd, msg, dnv, cv, gv, sem,
              spnum, spden):
    cid = lax.axis_index("c")
    sid = lax.axis_index("s")
    pltpu.sync_copy(consts, cv)
    pltpu.sync_copy(gmax_i, gv)
    li = lax.iota(jnp.int32, LN)
    u = [cv[cid, 0, pl.ds(q * LN, LN)] for q in range(2)]
    cc = [cv[cid, 1, pl.ds(q * LN, LN)] for q in range(2)]
    gval = gv[pl.ds(0, LN)]
    zv = jnp.zeros((LN,), jnp.float32)

    def zero_body(r, _):
        for q in range(2):
            msg[r, pl.ds(q * LN, LN)] = zv
        dnv[r, pl.ds(0, LN)] = zv
        return 0
    lax.fori_loop(0, CE, zero_body, 0)
    r0 = sid * STRIPE
    nfull = STRIPE // CE
    for t in range(nfull):
        pltpu.sync_copy(msg, spnum.at[pl.ds(r0 + t * CE, CE)])
        pltpu.sync_copy(dnv, spden.at[pl.ds(r0 + t * CE, CE)])
    rem = STRIPE - nfull * CE
    if rem:
        pltpu.sync_copy(msg.at[pl.ds(0, rem)],
                        spnum.at[pl.ds(r0 + nfull * CE, rem)])
        pltpu.sync_copy(dnv.at[pl.ds(0, rem)],
                        spden.at[pl.ds(r0 + nfull * CE, rem)])
    plsc.subcore_barrier()

    def chunk_body(ch, carry):
        row0 = sid * RPT2 + ch * CHUNK
        base = row0 * 128
        pltpu.sync_copy(dst2.at[pl.ds(row0, CHUNK)], idxd)
        pltpu.sync_copy(src2.at[pl.ds(row0, CHUNK)], idxs)
        pltpu.sync_copy(ef1.at[pl.ds(base, CE)], efv)
        pltpu.sync_copy(logits_i.at[pl.ds(base, CE)], lgv)
        cps = []
        for j in range(CHUNK):
            cps.append(pltpu.async_copy(
                Rh.at[cid].at[idxs.at[j]], rd.at[pl.ds(j * 128, 128)], sem))
        for cp in cps:
            cp.wait()

        def grp_body(g, _):
            e0 = g * LN
            eiv = e0 + li
            fv = efv[pl.ds(e0, LN)]
            lv = lgv[pl.ds(e0, LN)]
            ex = jnp.exp(lv - gval)
            for f in range(HD):
                colf = jnp.full((LN,), f, jnp.int32)
                rg = plsc.load_gather(rd, [eiv, colf])
                q, m = divmod(f, LN)
                mf = ex * (rg + fv * u[q][m] + cc[q][m])
                plsc.store_scatter(msg, [eiv, colf], mf)
            plsc.store_scatter(dnv, [eiv, jnp.zeros((LN,), jnp.int32)], ex)
            return 0

        lax.fori_loop(0, CE // LN, grp_body, 0)
        for j in range(CHUNK):
            pltpu.sync_copy(msg.at[pl.ds(j * 128, 128)],
                            spnum.at[idxd.at[j]], add=True)
            pltpu.sync_copy(dnv.at[pl.ds(j * 128, 128)],
                            spden.at[idxd.at[j]], add=True)
        return 0

    lax.fori_loop(0, NCHUNKS2, chunk_body, 0)
    plsc.subcore_barrier()
    pltpu.sync_copy(spnum.at[pl.ds(r0, STRIPE)],
                    num_o.at[cid, pl.ds(r0, STRIPE)])

    @pl.when(cid == 0)
    def _dump_den():
        pltpu.sync_copy(spden.at[pl.ds(r0, STRIPE)],
                        den_o.at[pl.ds(r0, STRIPE)])


_agg_call = pl.kernel(
    _agg_body,
    out_type=(jax.ShapeDtypeStruct((NC_SC, SP_ROWS, HD), jnp.float32),
              jax.ShapeDtypeStruct((SP_ROWS, LN), jnp.float32)),
    mesh=_mesh,
    compiler_params=_sc_params,
    scratch_types=(
        pltpu.VMEM((CHUNK, 128), jnp.int32),
        pltpu.VMEM((CHUNK, 128), jnp.int32),
        pltpu.VMEM((CE,), jnp.float32),
        pltpu.VMEM((CE,), jnp.float32),
        pltpu.VMEM((CE, HD), jnp.float32),
        pltpu.VMEM((CE, HD), jnp.float32),
        pltpu.VMEM((CE, LN), jnp.float32),
        pltpu.VMEM((NC_SC, 2, HD), jnp.float32),
        pltpu.VMEM((LN,), jnp.float32),
        pltpu.SemaphoreType.DMA,
        pltpu.VMEM_SHARED((SP_ROWS, HD), jnp.float32),
        pltpu.VMEM_SHARED((SP_ROWS, LN), jnp.float32),
    ),
)


BLK = 1000
GRID = N // BLK


def _mm(a, b):
    return jnp.dot(a, b, preferred_element_type=jnp.float32)


def _embed_body(cf, vf, csh, csc, wc1, bc1, wc2, bc2,
                vsh, vsc, wv1, bv1, wv2, bv2,
                wl1, bl1, wr1, br1, wl2, bl2,
                ce_o, ve_o, l1_o, r1_o, l2_o):
    c0 = (cf[...] - csh[...]) * csc[...]
    h = jnp.maximum(_mm(c0, wc1[...]) + bc1[...], 0.0)
    ce = jnp.maximum(_mm(h, wc2[...]) + bc2[...], 0.0)
    keep = lax.broadcasted_iota(jnp.int32, (1, 19), 1) == 2
    v0 = jnp.where(keep, vf[...], (vf[...] - vsh[...]) * vsc[...])
    h2 = jnp.maximum(_mm(v0, wv1[...]) + bv1[...], 0.0)
    ve = jnp.maximum(_mm(h2, wv2[...]) + bv2[...], 0.0)
    ce_o[...] = ce
    ve_o[...] = ve
    l1_o[...] = _mm(ce, wl1[...]) + bl1[...]
    r1_o[...] = _mm(ve, wr1[...]) + br1[...]
    l2_o[...] = _mm(ve, wl2[...]) + bl2[...]


def _row_spec(w):
    return pl.BlockSpec((BLK, w), lambda i: (i, 0))


def _full_spec(shape):
    return pl.BlockSpec(shape, lambda i: tuple(0 for _ in shape))


def _embed_call(cf, vf, csh, csc, wc1, bc1, wc2, bc2,
                vsh, vsc, wv1, bv1, wv2, bv2,
                wl1, bl1, wr1, br1, wl2, bl2):
    outs = tuple(jax.ShapeDtypeStruct((N, D), jnp.float32) for _ in range(5))
    args = (cf, vf, csh, csc, wc1, bc1, wc2, bc2,
            vsh, vsc, wv1, bv1, wv2, bv2, wl1, bl1, wr1, br1, wl2, bl2)
    in_specs = [_row_spec(5), _row_spec(19)] + [
        _full_spec(a.shape) for a in args[2:]]
    return pl.pallas_call(
        _embed_body,
        grid=(GRID,),
        in_specs=in_specs,
        out_specs=tuple(_row_spec(D) for _ in range(5)),
        out_shape=outs,
    )(*args)


def _mid_body(n0, d0, ce, wt, wb, bo, wr2, br2, r2_o):
    den = d0[:, 0:1] + 1e-16
    agg = n0[...] / den
    c2 = jnp.maximum(_mm(ce[...], wt[...]) + _mm(agg, wb[...]) + bo[...], 0.0)
    r2_o[...] = _mm(c2, wr2[...]) + br2[...]


def _mid_call(n0, d0, ce, wt, wb, bo, wr2, br2):
    args = (n0, d0, ce, wt, wb, bo, wr2, br2)
    in_specs = [_row_spec(D), _row_spec(LN),
                _row_spec(D)] + [_full_spec(a.shape) for a in args[3:]]
    return pl.pallas_call(
        _mid_body,
        grid=(GRID,),
        in_specs=in_specs,
        out_specs=_row_spec(D),
        out_shape=jax.ShapeDtypeStruct((N, D), jnp.float32),
    )(*args)


def _final_body(n0, d0, ve, wt, wb, bo, wo1, bo1, wo2, out_o):
    den = d0[:, 0:1] + 1e-16
    agg = n0[...] / den
    v2 = jnp.maximum(_mm(ve[...], wt[...]) + _mm(agg, wb[...]) + bo[...], 0.0)
    h = jnp.maximum(_mm(v2, wo1[...]) + bo1[...], 0.0)
    out_o[...] = _mm(h, wo2[...])


def _final_call(n0, d0, ve, wt, wb, bo, wo1, bo1, wo2):
    args = (n0, d0, ve, wt, wb, bo, wo1, bo1, wo2)
    in_specs = [_row_spec(D), _row_spec(LN),
                _row_spec(D)] + [_full_spec(a.shape) for a in args[3:]]
    return pl.pallas_call(
        _final_body,
        grid=(GRID,),
        in_specs=in_specs,
        out_specs=_row_spec(1),
        out_shape=jax.ShapeDtypeStruct((N, 1), jnp.float32),
    )(*args)


def kernel(constraint_features, edge_indices, edge_features, variable_features,
           variable_features_batch,
           cons_shift, cons_scale, W_c1, b_c1, W_c2, b_c2,
           edge_shift, edge_scale, W_e0, b_e0,
           var_shift, var_scale, W_v1, b_v1, W_v2, b_v2,
           g1_Wl, g1_bl, g1_Wr, g1_br, g1_We, g1_be, g1_att, g1_Wout, g1_bout,
           g2_Wl, g2_bl, g2_Wr, g2_br, g2_We, g2_be, g2_att, g2_Wout, g2_bout,
           W_o1, b_o1, W_o2):
    f32 = jnp.float32

    w0 = W_e0[0]
    b0 = b_e0 - edge_shift[0] * edge_scale[0] * w0
    w0s = edge_scale[0] * w0
    u1 = w0s @ g1_We
    c1 = b0 @ g1_We + g1_be
    u2 = w0s @ g2_We
    c2 = b0 @ g2_We + g2_be
    consts1 = jnp.stack([u1, c1, g1_att]).astype(f32)
    consts2 = jnp.stack([u2, c2, g2_att]).astype(f32)

    def _halves(u, c):
        return jnp.stack([jnp.stack([u[:HD], c[:HD]]),
                          jnp.stack([u[HD:], c[HD:]])]).astype(f32)

    consts1m = _halves(u1, c1)
    consts2m = _halves(u2, c2)

    pad = E_PAD - E
    src = edge_indices[1]
    dst = edge_indices[0]
    dst1_2d = jnp.pad(dst, (0, pad)).reshape(EROWS, 128)
    src1_2d = jnp.pad(src, (0, pad)).reshape(EROWS, 128)
    ef1 = jnp.pad(edge_features[:, 0], (0, pad)).astype(f32)

    ce_emb, ve_emb, L1, R1, L2 = _embed_call(
        constraint_features, variable_features,
        cons_shift.reshape(1, 5), cons_scale.reshape(1, 5),
        W_c1, b_c1.reshape(1, D), W_c2, b_c2.reshape(1, D),
        var_shift.reshape(1, 19), var_scale.reshape(1, 19),
        W_v1, b_v1.reshape(1, D), W_v2, b_v2.reshape(1, D),
        g1_Wl, g1_bl.reshape(1, D), g1_Wr, g1_br.reshape(1, D),
        g2_Wl, g2_bl.reshape(1, D))

    def _stack_halves(R):
        return jnp.stack([R[:, :HD], R[:, HD:]])

    logits1, maxes1 = _logits_call(L1, R1, dst1_2d, src1_2d, ef1, consts1)
    gmax1 = jnp.broadcast_to(jnp.max(maxes1), (LN,)).astype(f32)
    num1, den1 = _agg_call(_stack_halves(R1), dst1_2d, src1_2d, ef1,
                           logits1, gmax1, consts1m)
    num1f = jnp.concatenate([num1[0, :N], num1[1, :N]], axis=1)

    R2 = _mid_call(num1f, den1[:N],
                   ce_emb, g1_Wout[:D], g1_Wout[D:], g1_bout.reshape(1, D),
                   g2_Wr, g2_br.reshape(1, D))

    logits2, maxes2 = _logits_call(L2, R2, src1_2d, dst1_2d, ef1, consts2)
    gmax2 = jnp.broadcast_to(jnp.max(maxes2), (LN,)).astype(f32)
    num2, den2 = _agg_call(_stack_halves(R2), src1_2d, dst1_2d, ef1,
                           logits2, gmax2, consts2m)
    num2f = jnp.concatenate([num2[0, :N], num2[1, :N]], axis=1)

    out = _final_call(num2f, den2[:N],
                      ve_emb, g2_Wout[:D], g2_Wout[D:], g2_bout.reshape(1, D),
                      W_o1, b_o1.reshape(1, D), W_o2)
    return out[:, 0]
